# baseline (speedup 1.0000x reference)
import functools

import jax
import jax.numpy as jnp
from jax import lax
from jax.experimental import pallas as pl
from jax.experimental.pallas import tpu as pltpu
from jax.experimental.pallas import tpu_sc as plsc

N_NODES = 10000
N_EDGES = 320000
D = 128
H = 64

NW = 32
EP = N_EDGES // NW
C = 80
CH = EP // C
L = 16
G = C // L


def _gate_c_body(sd_ref, b2_ref, embed_ref, w1_ref, b1_ref, noise_ref,
                 gate_ref, c_ref):
    s = sd_ref[0]
    d_ = sd_ref[1]
    es = embed_ref[pl.ds(s, 1), :]
    ed = embed_ref[pl.ds(d_, 1), :]
    cvec = (jnp.dot(es, w1_ref[2 * D:3 * D, :], preferred_element_type=jnp.float32)
            + jnp.dot(ed, w1_ref[3 * D:4 * D, :], preferred_element_type=jnp.float32)
            + b1_ref[...])
    c_ref[...] = cvec
    nz = noise_ref[...]
    gate_ref[...] = jnp.log(nz) - jnp.log(1.0 - nz) + b2_ref[0]


def _pq_body(embed_ref, w1_ref, c_ref, pp_ref, qq_ref):
    emb = embed_ref[...]
    pp_ref[...] = (jnp.dot(emb, w1_ref[0:D, :], preferred_element_type=jnp.float32)
                   + c_ref[...])
    qq_ref[...] = jnp.dot(emb, w1_ref[D:2 * D, :], preferred_element_type=jnp.float32)


def _sc_body(pp_hbm, qq_hbm, col_hbm, row_hbm, gate_hbm, w2_hbm, out_hbm,
             colv, rowv, gv, ov, pv0, qv0, pv1, qv1, w2x2, sem0, sem1):
    wid = lax.axis_index("c") * 16 + lax.axis_index("s")
    pltpu.sync_copy(col_hbm.at[wid], colv)
    pltpu.sync_copy(row_hbm.at[wid], rowv)
    pltpu.sync_copy(gate_hbm.at[wid], gv)
    pltpu.sync_copy(w2_hbm, w2x2.at[pl.ds(0, H)])

    def issue(ci, pv, qv, sem):
        pltpu.async_copy(pp_hbm.at[colv.at[ci]], pv, sem)
        pltpu.async_copy(qq_hbm.at[rowv.at[ci]], qv, sem)

    def drain(pv, qv, sem):
        pltpu.make_async_copy(pp_hbm.at[colv.at[0]], pv, sem).wait()
        pltpu.make_async_copy(qq_hbm.at[rowv.at[0]], qv, sem).wait()

    iota = lax.iota(jnp.int32, L)
    w2v = [w2x2[pl.ds(seg * L, L)] for seg in range(H // L)]
    shuf = [iota ^ sh for sh in (8, 4, 2, 1)]

    def compute(ci, pv, qv):
        base = ci * C

        @plsc.parallel_loop(0, C, L)
        def _(eb):
            acc = None
            for k in range(L):
                m = None
                for seg in range(H // L):
                    ps = pv[eb + k, pl.ds(seg * L, L)]
                    qs = qv[eb + k, pl.ds(seg * L, L)]
                    t = jnp.maximum(ps + qs, 0.0) * w2v[seg]
                    m = t if m is None else m + t
                for ix in shuf:
                    m = m + m.at[ix].get(mode="promise_in_bounds")
                acc = m if acc is None else jnp.where(iota == k, m, acc)
            sl = pl.ds(base + eb, L)
            x = acc + gv[sl]
            ov[sl] = 1.0 / (1.0 + jnp.exp(-x))

    issue(0, pv0, qv0, sem0)

    def body(i, carry):
        ci0 = 2 * i
        issue(ci0 + 1, pv1, qv1, sem1)
        drain(pv0, qv0, sem0)
        compute(ci0, pv0, qv0)

        @pl.when(ci0 + 2 < CH)
        def _():
            issue(ci0 + 2, pv0, qv0, sem0)

        drain(pv1, qv1, sem1)
        compute(ci0 + 1, pv1, qv1)
        return carry

    lax.fori_loop(0, CH // 2, body, 0)

    drain(pv0, qv0, sem0)
    compute(CH - 1, pv0, qv0)
    pltpu.sync_copy(ov, out_hbm.at[wid])


def kernel(embed, edge_index, W1, b1, W2, b2, noise, src, dst):
    sd = jnp.stack([jnp.asarray(src, jnp.int32), jnp.asarray(dst, jnp.int32)])
    noise2 = noise.reshape(N_EDGES // D, D)

    gate2, cvec = pl.pallas_call(
        _gate_c_body,
        out_shape=(
            jax.ShapeDtypeStruct((N_EDGES // D, D), jnp.float32),
            jax.ShapeDtypeStruct((1, H), jnp.float32),
        ),
        in_specs=[
            pl.BlockSpec(memory_space=pltpu.SMEM),
            pl.BlockSpec(memory_space=pltpu.SMEM),
            pl.BlockSpec(memory_space=pltpu.VMEM),
            pl.BlockSpec(memory_space=pltpu.VMEM),
            pl.BlockSpec(memory_space=pltpu.VMEM),
            pl.BlockSpec(memory_space=pltpu.VMEM),
        ],
        out_specs=(
            pl.BlockSpec(memory_space=pltpu.VMEM),
            pl.BlockSpec(memory_space=pltpu.VMEM),
        ),
    )(sd, b2, embed, W1, b1.reshape(1, H), noise2)

    RB = 1000
    pp, qq = pl.pallas_call(
        _pq_body,
        grid=(N_NODES // RB,),
        in_specs=[
            pl.BlockSpec((RB, D), lambda i: (i, 0)),
            pl.BlockSpec((4 * D, H), lambda i: (0, 0)),
            pl.BlockSpec((1, H), lambda i: (0, 0)),
        ],
        out_specs=(
            pl.BlockSpec((RB, H), lambda i: (i, 0)),
            pl.BlockSpec((RB, H), lambda i: (i, 0)),
        ),
        out_shape=(
            jax.ShapeDtypeStruct((N_NODES, H), jnp.float32),
            jax.ShapeDtypeStruct((N_NODES, H), jnp.float32),
        ),
    )(embed, W1, cvec)

    col3 = edge_index[0].reshape(NW, CH, C)
    row3 = edge_index[1].reshape(NW, CH, C)
    gate_w = gate2.reshape(NW, EP)

    mesh = plsc.VectorSubcoreMesh(core_axis_name="c", subcore_axis_name="s")
    sc = functools.partial(
        pl.kernel,
        mesh=mesh,
        out_type=jax.ShapeDtypeStruct((NW, EP), jnp.float32),
        scratch_types=[
            pltpu.VMEM((CH, C), jnp.int32),
            pltpu.VMEM((CH, C), jnp.int32),
            pltpu.VMEM((EP,), jnp.float32),
            pltpu.VMEM((EP,), jnp.float32),
            pltpu.VMEM((C, H), jnp.float32),
            pltpu.VMEM((C, H), jnp.float32),
            pltpu.VMEM((C, H), jnp.float32),
            pltpu.VMEM((C, H), jnp.float32),
            pltpu.VMEM((2 * H,), jnp.float32),
            pltpu.SemaphoreType.DMA,
            pltpu.SemaphoreType.DMA,
        ],
        compiler_params=pltpu.CompilerParams(
            needs_layout_passes=False, use_tc_tiling_on_sc=False),
    )(_sc_body)

    out_w = sc(pp, qq, col3, row3, gate_w, W2.reshape(H))
    return out_w.reshape(N_EDGES)


# speedup vs baseline: 1.6084x; 1.6084x over previous
import functools

import jax
import jax.numpy as jnp
from jax import lax
from jax.experimental import pallas as pl
from jax.experimental.pallas import tpu as pltpu
from jax.experimental.pallas import tpu_sc as plsc

N_NODES = 10000
N_EDGES = 320000
D = 128
H = 64

NW = 32
EP = N_EDGES // NW
C = 80
CH = EP // C
L = 16
G = C // L


def _gate_c_body(sd_ref, b2_ref, embed_ref, w1_ref, b1_ref, noise_ref,
                 gate_ref, c_ref):
    s = sd_ref[0]
    d_ = sd_ref[1]
    es = embed_ref[pl.ds(s, 1), :]
    ed = embed_ref[pl.ds(d_, 1), :]
    cvec = (jnp.dot(es, w1_ref[2 * D:3 * D, :], preferred_element_type=jnp.float32)
            + jnp.dot(ed, w1_ref[3 * D:4 * D, :], preferred_element_type=jnp.float32)
            + b1_ref[...])
    c_ref[...] = cvec
    nz = noise_ref[...]
    gate_ref[...] = jnp.log(nz) - jnp.log(1.0 - nz) + b2_ref[0]


def _pq_body(embed_ref, w1_ref, c_ref, pp_ref, qq_ref):
    emb = embed_ref[...]
    pp_ref[...] = (jnp.dot(emb, w1_ref[0:D, :], preferred_element_type=jnp.float32)
                   + c_ref[...])
    qq_ref[...] = jnp.dot(emb, w1_ref[D:2 * D, :], preferred_element_type=jnp.float32)


def _sc_body(pp_hbm, qq_hbm, col_hbm, row_hbm, gate_hbm, w2_hbm, out_hbm,
             colv, rowv, gv, ov, pv0, qv0, pv1, qv1, w2x2, sem0, sem1):
    wid = lax.axis_index("c") * 16 + lax.axis_index("s")
    pltpu.sync_copy(col_hbm.at[wid], colv)
    pltpu.sync_copy(row_hbm.at[wid], rowv)
    pltpu.sync_copy(gate_hbm.at[wid], gv)
    pltpu.sync_copy(w2_hbm, w2x2.at[pl.ds(0, H)])

    def issue(ci, pv, qv, sem):
        pltpu.async_copy(pp_hbm.at[colv.at[ci]], pv, sem)
        pltpu.async_copy(qq_hbm.at[rowv.at[ci]], qv, sem)

    def drain(pv, qv, sem):
        pltpu.make_async_copy(pp_hbm.at[colv.at[0]], pv, sem).wait()
        pltpu.make_async_copy(qq_hbm.at[rowv.at[0]], qv, sem).wait()

    iota = lax.iota(jnp.int32, L)
    w2v = [w2x2[pl.ds(seg * L, L)] for seg in range(H // L)]
    shuf = [iota ^ sh for sh in (8, 4, 2, 1)]

    def compute(ci, pv, qv):
        base = ci * C
        for g in range(G):
            acc = None
            for k in range(L):
                e = g * L + k
                m = None
                for seg in range(H // L):
                    ps = pv[e, pl.ds(seg * L, L)]
                    qs = qv[e, pl.ds(seg * L, L)]
                    t = jnp.maximum(ps + qs, 0.0) * w2v[seg]
                    m = t if m is None else m + t
                for ix in shuf:
                    m = m + m.at[ix].get(mode="promise_in_bounds")
                acc = m if acc is None else jnp.where(iota == k, m, acc)
            sl = pl.ds(base + g * L, L)
            x = acc + gv[sl]
            ov[sl] = 1.0 / (1.0 + jnp.exp(-x))

    issue(0, pv0, qv0, sem0)

    def body(i, carry):
        ci0 = 2 * i
        issue(ci0 + 1, pv1, qv1, sem1)
        drain(pv0, qv0, sem0)
        compute(ci0, pv0, qv0)

        @pl.when(ci0 + 2 < CH)
        def _():
            issue(ci0 + 2, pv0, qv0, sem0)

        drain(pv1, qv1, sem1)
        compute(ci0 + 1, pv1, qv1)
        return carry

    lax.fori_loop(0, CH // 2, body, 0)

    drain(pv0, qv0, sem0)
    compute(CH - 1, pv0, qv0)
    pltpu.sync_copy(ov, out_hbm.at[wid])


def kernel(embed, edge_index, W1, b1, W2, b2, noise, src, dst):
    sd = jnp.stack([jnp.asarray(src, jnp.int32), jnp.asarray(dst, jnp.int32)])
    noise2 = noise.reshape(N_EDGES // D, D)

    gate2, cvec = pl.pallas_call(
        _gate_c_body,
        out_shape=(
            jax.ShapeDtypeStruct((N_EDGES // D, D), jnp.float32),
            jax.ShapeDtypeStruct((1, H), jnp.float32),
        ),
        in_specs=[
            pl.BlockSpec(memory_space=pltpu.SMEM),
            pl.BlockSpec(memory_space=pltpu.SMEM),
            pl.BlockSpec(memory_space=pltpu.VMEM),
            pl.BlockSpec(memory_space=pltpu.VMEM),
            pl.BlockSpec(memory_space=pltpu.VMEM),
            pl.BlockSpec(memory_space=pltpu.VMEM),
        ],
        out_specs=(
            pl.BlockSpec(memory_space=pltpu.VMEM),
            pl.BlockSpec(memory_space=pltpu.VMEM),
        ),
    )(sd, b2, embed, W1, b1.reshape(1, H), noise2)

    RB = 1000
    pp, qq = pl.pallas_call(
        _pq_body,
        grid=(N_NODES // RB,),
        in_specs=[
            pl.BlockSpec((RB, D), lambda i: (i, 0)),
            pl.BlockSpec((4 * D, H), lambda i: (0, 0)),
            pl.BlockSpec((1, H), lambda i: (0, 0)),
        ],
        out_specs=(
            pl.BlockSpec((RB, H), lambda i: (i, 0)),
            pl.BlockSpec((RB, H), lambda i: (i, 0)),
        ),
        out_shape=(
            jax.ShapeDtypeStruct((N_NODES, H), jnp.float32),
            jax.ShapeDtypeStruct((N_NODES, H), jnp.float32),
        ),
    )(embed, W1, cvec)

    col3 = edge_index[0].reshape(NW, CH, C)
    row3 = edge_index[1].reshape(NW, CH, C)
    gate_w = gate2.reshape(NW, EP)

    mesh = plsc.VectorSubcoreMesh(core_axis_name="c", subcore_axis_name="s")
    sc = functools.partial(
        pl.kernel,
        mesh=mesh,
        out_type=jax.ShapeDtypeStruct((NW, EP), jnp.float32),
        scratch_types=[
            pltpu.VMEM((CH, C), jnp.int32),
            pltpu.VMEM((CH, C), jnp.int32),
            pltpu.VMEM((EP,), jnp.float32),
            pltpu.VMEM((EP,), jnp.float32),
            pltpu.VMEM((C, H), jnp.float32),
            pltpu.VMEM((C, H), jnp.float32),
            pltpu.VMEM((C, H), jnp.float32),
            pltpu.VMEM((C, H), jnp.float32),
            pltpu.VMEM((2 * H,), jnp.float32),
            pltpu.SemaphoreType.DMA,
            pltpu.SemaphoreType.DMA,
        ],
        compiler_params=pltpu.CompilerParams(
            needs_layout_passes=False, use_tc_tiling_on_sc=False),
    )(_sc_body)

    out_w = sc(pp, qq, col3, row3, gate_w, W2.reshape(H))
    return out_w.reshape(N_EDGES)
